# baseline (speedup 1.0000x reference)
import jax
import jax.numpy as jnp
from jax import lax
from jax.experimental import pallas as pl
from jax.experimental.pallas import tpu as pltpu
from jax.experimental.pallas import tpu_sc as plsc

F = 26
B = 4096
VOCAB = 100000
FED = 32
N = 4
D = 32
HID = 208
T = 200
NITEMS = 1000000
SED = 128

NC = 2
NS = 16
NW = NC * NS
LANES = 16

_MESH = plsc.VectorSubcoreMesh(core_axis_name="c", subcore_axis_name="s")
_SC_UNTILED = pltpu.CompilerParams(use_tc_tiling_on_sc=False)


BPW = B // NW


def _nonseq_body(idx_hbm, tables_hbm, out_hbm, idx_v, rows_v, sem):
    wid = lax.axis_index("s") * NC + lax.axis_index("c")
    b0 = wid * BPW
    for f in range(F):
        pltpu.sync_copy(idx_hbm.at[pl.ds(f * B + b0, BPW)], idx_v)
        for c in range(BPW // LANES):
            sl = pl.ds(c * LANES, LANES)
            idx_v[sl] = idx_v[sl] + f * VOCAB
        pltpu.async_copy(tables_hbm.at[idx_v], rows_v, sem).wait()
        pltpu.sync_copy(rows_v, out_hbm.at[f, pl.ds(b0, BPW)])


_nonseq_gather = pl.kernel(
    _nonseq_body,
    out_type=jax.ShapeDtypeStruct((F, B, FED), jnp.float32),
    mesh=_MESH,
    scratch_types=[
        pltpu.VMEM((BPW,), jnp.int32),
        pltpu.VMEM((BPW, FED), jnp.float32),
        pltpu.SemaphoreType.DMA,
    ],
    compiler_params=_SC_UNTILED,
)


BLK = 512


def _head_proj_body(e_ref, w_ref, o_ref):
    accs = [jnp.zeros((BLK, D), jnp.float32) for _ in range(N)]
    for f in range(F):
        e_f = e_ref[f]
        j0 = f * FED
        for n in range(j0 // HID, (j0 + FED - 1) // HID + 1):
            lo = max(j0, n * HID)
            hi = min(j0 + FED, (n + 1) * HID)
            w = w_ref[n, :, lo - n * HID:hi - n * HID]
            ef = e_f[:, lo - j0:hi - j0]
            accs[n] = accs[n] + lax.dot_general(
                ef, w, (((1,), (1,)), ((), ())),
                preferred_element_type=jnp.float32)
    o_ref[:, :] = jnp.concatenate(accs, axis=1)


def _head_proj(e, head_W):
    return pl.pallas_call(
        _head_proj_body,
        grid=(B // BLK,),
        in_specs=[
            pl.BlockSpec((F, BLK, FED), lambda i: (0, i, 0)),
            pl.BlockSpec((N, D, HID), lambda i: (0, 0, 0)),
        ],
        out_specs=pl.BlockSpec((BLK, N * D), lambda i: (i, 0)),
        out_shape=jax.ShapeDtypeStruct((B, N * D), jnp.float32),
    )(e, head_W)


ROWS = B * T
RPW = ROWS // NW
CHUNKS = RPW // T
SPLIT = 104
NBUF = 4


def _seq_body(idx_hbm, item_hbm, pos_hbm, out_hbm, *rest):
    idx_bufs = rest[0:NBUF]
    row_bufs = rest[NBUF:2 * NBUF]
    pos_v = rest[2 * NBUF]
    gsems = rest[2 * NBUF + 1:3 * NBUF + 1]
    wsems = rest[3 * NBUF + 1:4 * NBUF + 1]

    wid = lax.axis_index("s") * NC + lax.axis_index("c")
    b0 = wid * CHUNKS
    pltpu.sync_copy(pos_hbm, pos_v)

    def start_gather(c, j):
        pltpu.sync_copy(idx_hbm.at[pl.ds((b0 + c) * T, T)], idx_bufs[j])
        pltpu.async_copy(item_hbm.at[idx_bufs[j].at[pl.ds(0, SPLIT)]],
                         row_bufs[j].at[pl.ds(0, SPLIT)], gsems[j])
        pltpu.async_copy(item_hbm.at[idx_bufs[j].at[pl.ds(SPLIT, T - SPLIT)]],
                         row_bufs[j].at[pl.ds(SPLIT, T - SPLIT)], gsems[j])

    def wait_gather(j):
        pltpu.make_async_copy(item_hbm.at[idx_bufs[j]], row_bufs[j],
                              gsems[j]).wait()

    def wait_wb(c, j):
        pltpu.make_async_copy(row_bufs[j], out_hbm.at[b0 + c], wsems[j]).wait()

    start_gather(0, 0)

    def body(g, carry):
        for j in range(NBUF):
            c = g * NBUF + j
            jn = (j + 1) % NBUF
            @pl.when(c >= NBUF - 1)
            def _():
                wait_wb(c - (NBUF - 1), jn)

            @pl.when(c + 1 < CHUNKS)
            def _():
                start_gather(c + 1, jn)

            wait_gather(j)

            def add_row(r, c2):
                for s in range(SED // LANES):
                    sl = pl.ds(s * LANES, LANES)
                    row_bufs[j][r, sl] = row_bufs[j][r, sl] + pos_v[r, sl]
                return c2

            lax.fori_loop(0, T, add_row, 0)
            pltpu.async_copy(row_bufs[j], out_hbm.at[b0 + c], wsems[j])
        return carry

    lax.fori_loop(0, CHUNKS // NBUF, body, 0)
    for j in range(1, NBUF):
        wait_wb(CHUNKS - NBUF + j, j)


_seq_gather = pl.kernel(
    _seq_body,
    out_type=jax.ShapeDtypeStruct((B, T, SED), jnp.float32),
    mesh=_MESH,
    scratch_types=(
        [pltpu.VMEM((T,), jnp.int32)] * NBUF
        + [pltpu.VMEM((T, SED), jnp.float32)] * NBUF
        + [pltpu.VMEM((T, SED), jnp.float32)]
        + [pltpu.SemaphoreType.DMA] * (2 * NBUF)
    ),
)


VA, VB = 400, 250
SBLK = 40


def _tables_t_body(t_ref, o_ref):
    x = t_ref[0].reshape(FED, SBLK * VB)
    o_ref[0] = x.T.reshape(SBLK, VB, FED)


def _tables_transpose(ft4):
    return pl.pallas_call(
        _tables_t_body,
        grid=(F, VA // SBLK),
        in_specs=[pl.BlockSpec((1, FED, SBLK, VB), lambda f, j: (f, 0, j, 0))],
        out_specs=pl.BlockSpec((1, SBLK, VB, FED), lambda f, j: (f, j, 0, 0)),
        out_shape=jax.ShapeDtypeStruct((F, VA, VB, FED), jnp.float32),
    )(ft4)


def kernel(non_seq_indices, seq_features, feature_tables, head_W, item_table, pos_table):
    ft4 = feature_tables.transpose(0, 2, 1).reshape(F, FED, VA, VB)
    tables_flat = _tables_transpose(ft4).reshape(F * VOCAB, FED)
    e = _nonseq_gather(non_seq_indices.reshape(F * B), tables_flat)
    x = _head_proj(e, head_W).reshape(B, N, D)
    seq = _seq_gather(seq_features.reshape(ROWS), item_table, pos_table)
    return x, seq


# speedup vs baseline: 1.5199x; 1.5199x over previous
import jax
import jax.numpy as jnp
from jax import lax
from jax.experimental import pallas as pl
from jax.experimental.pallas import tpu as pltpu
from jax.experimental.pallas import tpu_sc as plsc

F = 26
B = 4096
VOCAB = 100000
FED = 32
N = 4
D = 32
HID = 208
T = 200
NITEMS = 1000000
SED = 128

NC = 2
NS = 16
NW = NC * NS
LANES = 16

_MESH = plsc.VectorSubcoreMesh(core_axis_name="c", subcore_axis_name="s")
_SC_UNTILED = pltpu.CompilerParams(use_tc_tiling_on_sc=False)


BPW = B // NW


def _nonseq_body(idx_hbm, tables_hbm, out_hbm, idx_v, rows_v, sem):
    wid = lax.axis_index("s") * NC + lax.axis_index("c")
    b0 = wid * BPW
    for f in range(F):
        pltpu.sync_copy(idx_hbm.at[pl.ds(f * B + b0, BPW)], idx_v)
        for c in range(BPW // LANES):
            sl = pl.ds(c * LANES, LANES)
            idx_v[sl] = idx_v[sl] + f * VOCAB
        pltpu.async_copy(tables_hbm.at[idx_v], rows_v, sem).wait()
        pltpu.sync_copy(rows_v, out_hbm.at[f, pl.ds(b0, BPW)])


_nonseq_gather = pl.kernel(
    _nonseq_body,
    out_type=jax.ShapeDtypeStruct((F, B, FED), jnp.float32),
    mesh=_MESH,
    scratch_types=[
        pltpu.VMEM((BPW,), jnp.int32),
        pltpu.VMEM((BPW, FED), jnp.float32),
        pltpu.SemaphoreType.DMA,
    ],
    compiler_params=_SC_UNTILED,
)


BLK = 512


def _head_proj_body(e_ref, w_ref, o_ref):
    accs = [jnp.zeros((BLK, D), jnp.float32) for _ in range(N)]
    for f in range(F):
        e_f = e_ref[f]
        j0 = f * FED
        for n in range(j0 // HID, (j0 + FED - 1) // HID + 1):
            lo = max(j0, n * HID)
            hi = min(j0 + FED, (n + 1) * HID)
            w = w_ref[n, :, lo - n * HID:hi - n * HID]
            ef = e_f[:, lo - j0:hi - j0]
            accs[n] = accs[n] + lax.dot_general(
                ef, w, (((1,), (1,)), ((), ())),
                preferred_element_type=jnp.float32)
    o_ref[:, :] = jnp.concatenate(accs, axis=1)


def _head_proj(e, head_W):
    return pl.pallas_call(
        _head_proj_body,
        grid=(B // BLK,),
        in_specs=[
            pl.BlockSpec((F, BLK, FED), lambda i: (0, i, 0)),
            pl.BlockSpec((N, D, HID), lambda i: (0, 0, 0)),
        ],
        out_specs=pl.BlockSpec((BLK, N * D), lambda i: (i, 0)),
        out_shape=jax.ShapeDtypeStruct((B, N * D), jnp.float32),
    )(e, head_W)


ROWS = B * T
RPW = ROWS // NW
CHUNKS = RPW // T
SPLIT = 104
NBUF = 4


def _seq_body(idx_hbm, item_hbm, pos_hbm, out_hbm, *rest):
    idx_bufs = rest[0:NBUF]
    row_bufs = rest[NBUF:2 * NBUF]
    pos_v = rest[2 * NBUF]
    gsems = rest[2 * NBUF + 1:3 * NBUF + 1]
    wsems = rest[3 * NBUF + 1:4 * NBUF + 1]

    wid = lax.axis_index("s") * NC + lax.axis_index("c")
    b0 = wid * CHUNKS
    pltpu.sync_copy(pos_hbm, pos_v)

    def start_gather(c, j):
        pltpu.sync_copy(idx_hbm.at[pl.ds((b0 + c) * T, T)], idx_bufs[j])
        pltpu.async_copy(item_hbm.at[idx_bufs[j].at[pl.ds(0, SPLIT)]],
                         row_bufs[j].at[pl.ds(0, SPLIT)], gsems[j])
        pltpu.async_copy(item_hbm.at[idx_bufs[j].at[pl.ds(SPLIT, T - SPLIT)]],
                         row_bufs[j].at[pl.ds(SPLIT, T - SPLIT)], gsems[j])

    def wait_gather(j):
        pltpu.make_async_copy(item_hbm.at[idx_bufs[j]], row_bufs[j],
                              gsems[j]).wait()

    def wait_wb(c, j):
        pltpu.make_async_copy(row_bufs[j], out_hbm.at[b0 + c], wsems[j]).wait()

    start_gather(0, 0)

    def body(g, carry):
        for j in range(NBUF):
            c = g * NBUF + j
            jn = (j + 1) % NBUF
            @pl.when(c >= NBUF - 1)
            def _():
                wait_wb(c - (NBUF - 1), jn)

            @pl.when(c + 1 < CHUNKS)
            def _():
                start_gather(c + 1, jn)

            wait_gather(j)

            def add_row(r, c2):
                for s in range(SED // LANES):
                    sl = pl.ds(s * LANES, LANES)
                    row_bufs[j][r, sl] = row_bufs[j][r, sl] + pos_v[r, sl]
                return c2

            lax.fori_loop(0, T, add_row, 0)
            pltpu.async_copy(row_bufs[j], out_hbm.at[b0 + c], wsems[j])
        return carry

    lax.fori_loop(0, CHUNKS // NBUF, body, 0)
    for j in range(1, NBUF):
        wait_wb(CHUNKS - NBUF + j, j)


_seq_gather = pl.kernel(
    _seq_body,
    out_type=jax.ShapeDtypeStruct((B, T, SED), jnp.float32),
    mesh=_MESH,
    scratch_types=(
        [pltpu.VMEM((T,), jnp.int32)] * NBUF
        + [pltpu.VMEM((T, SED), jnp.float32)] * NBUF
        + [pltpu.VMEM((T, SED), jnp.float32)]
        + [pltpu.SemaphoreType.DMA] * (2 * NBUF)
    ),
)


def kernel(non_seq_indices, seq_features, feature_tables, head_W, item_table, pos_table):
    tables_flat = feature_tables.reshape(F * VOCAB, FED)
    e = _nonseq_gather(non_seq_indices.reshape(F * B), tables_flat)
    x = _head_proj(e, head_W).reshape(B, N, D)
    seq = _seq_gather(seq_features.reshape(ROWS), item_table, pos_table)
    return x, seq
